# Initial kernel scaffold; baseline (speedup 1.0000x reference)
import jax
import jax.numpy as jnp
from jax.experimental import pallas as pl


def kernel(x, mem_kv, W_attn, W_proj, gate_bias):
    raise NotImplementedError("write your pallas kernel here")


# speedup vs baseline: 39.4346x; 39.4346x over previous
import functools

import jax
import jax.numpy as jnp
from jax import lax
from jax.experimental import pallas as pl
from jax.experimental.pallas import tpu as pltpu
from jax.experimental.pallas import tpu_sc as plsc


def _qkv_kernel(x_ref, w_ref, o_ref):
    o_ref[0] = jnp.dot(x_ref[...], w_ref[0],
                       preferred_element_type=jnp.float32)


def _qkv_proj(xf, w_heads):
    t, ne = xf.shape
    nh3 = w_heads.shape[0]
    c = w_heads.shape[2]
    return pl.pallas_call(
        _qkv_kernel,
        grid=(nh3,),
        in_specs=[
            pl.BlockSpec((t, ne), lambda j: (0, 0)),
            pl.BlockSpec((1, ne, c), lambda j: (j, 0, 0)),
        ],
        out_specs=pl.BlockSpec((1, t, c), lambda j: (j, 0, 0)),
        out_shape=jax.ShapeDtypeStruct((nh3, t, c), jnp.float32),
        compiler_params=pltpu.CompilerParams(
            dimension_semantics=("parallel",)),
    )(xf, w_heads)


def _causal_kernel(q_ref, k_ref, v_ref, o_ref, *, tq, c):
    i = pl.program_id(1)
    q = q_ref[0]
    k = k_ref[0]
    v = v_ref[0]
    s = lax.dot_general(q, k, (((1,), (1,)), ((), ())),
                        preferred_element_type=jnp.float32)
    s = s * (1.0 / (c ** 0.5))
    row = lax.broadcasted_iota(jnp.int32, s.shape, 0) + i * tq
    col = lax.broadcasted_iota(jnp.int32, s.shape, 1)
    s = jnp.where(col <= row, s, -1e30)
    m = jnp.max(s, axis=1, keepdims=True)
    p = jnp.exp(s - m)
    denom = jnp.sum(p, axis=1, keepdims=True)
    y = lax.dot_general(p, v, (((1,), (0,)), ((), ())),
                        preferred_element_type=jnp.float32)
    o_ref[0] = y / denom


def _causal_attn(qkv_h, h, tq=512):
    t = qkv_h.shape[1]
    c = qkv_h.shape[2]
    kern = functools.partial(_causal_kernel, tq=tq, c=c)
    return pl.pallas_call(
        kern,
        grid=(h, t // tq),
        in_specs=[
            pl.BlockSpec((1, tq, c), lambda hh, i: (hh, i, 0)),
            pl.BlockSpec((1, t, c), lambda hh, i: (h + hh, 0, 0)),
            pl.BlockSpec((1, t, c), lambda hh, i: (2 * h + hh, 0, 0)),
        ],
        out_specs=pl.BlockSpec((1, tq, c), lambda hh, i: (hh, i, 0)),
        out_shape=jax.ShapeDtypeStruct((h, t, c), jnp.float32),
        compiler_params=pltpu.CompilerParams(
            dimension_semantics=("parallel", "parallel")),
    )(qkv_h, qkv_h, qkv_h)


def _knn_kernel(q_ref, memk_ref, topi_ref,
                v0, v1, v2, i0, i1, i2, *, mt, t, h, c):
    step = pl.program_id(0)
    nsteps = pl.num_programs(0)

    @pl.when(step == 0)
    def _init():
        neg = jnp.full((1, t), -jnp.inf, jnp.float32)
        zero = jnp.zeros((1, t), jnp.int32)
        v0[...] = neg
        v1[...] = neg
        v2[...] = neg
        i0[...] = zero
        i1[...] = zero
        i2[...] = zero

    s = jnp.zeros((mt, t), jnp.float32)
    for hh in range(h):
        k_h = memk_ref[:, 0, pl.ds(hh * c, c)]
        q_h = q_ref[hh]
        s = s + lax.dot_general(k_h, q_h, (((1,), (1,)), ((), ())),
                                preferred_element_type=jnp.float32)
    rows = lax.broadcasted_iota(jnp.int32, (mt, t), 0)
    base = step * mt
    for _ in range(3):
        m = jnp.max(s, axis=0, keepdims=True)
        cand = jnp.where(s == m, rows, mt)
        am = jnp.min(cand, axis=0, keepdims=True)
        s = jnp.where(rows == am, -jnp.inf, s)
        ci = am + base
        c0, c1, c2 = v0[...], v1[...], v2[...]
        j0, j1, j2 = i0[...], i1[...], i2[...]
        gt0 = m > c0
        gt1 = m > c1
        gt2 = m > c2
        v0[...] = jnp.where(gt0, m, c0)
        i0[...] = jnp.where(gt0, ci, j0)
        v1[...] = jnp.where(gt0, c0, jnp.where(gt1, m, c1))
        i1[...] = jnp.where(gt0, j0, jnp.where(gt1, ci, j1))
        v2[...] = jnp.where(gt1, c1, jnp.where(gt2, m, c2))
        i2[...] = jnp.where(gt1, j1, jnp.where(gt2, ci, j2))

    @pl.when(step == nsteps - 1)
    def _emit():
        topi_ref[...] = jnp.zeros((8, t), jnp.int32)
        topi_ref[0:1, :] = i0[...]
        topi_ref[1:2, :] = i1[...]
        topi_ref[2:3, :] = i2[...]


def _knn_top3(qkv_h, mem3, h, c, mt=1024):
    t = qkv_h.shape[1]
    ne = h * c
    m = mem3.shape[0]
    kern = functools.partial(_knn_kernel, mt=mt, t=t, h=h, c=c)
    return pl.pallas_call(
        kern,
        grid=(m // mt,),
        in_specs=[
            pl.BlockSpec((h, t, c), lambda i: (0, 0, 0)),
            pl.BlockSpec((mt, 2, ne), lambda i: (i, 0, 0)),
        ],
        out_specs=pl.BlockSpec((8, t), lambda i: (0, 0)),
        out_shape=jax.ShapeDtypeStruct((8, t), jnp.int32),
        scratch_shapes=[pltpu.VMEM((1, t), jnp.float32)] * 3
        + [pltpu.VMEM((1, t), jnp.int32)] * 3,
        compiler_params=pltpu.CompilerParams(
            dimension_semantics=("arbitrary",)),
    )(qkv_h, mem3)


def _gather_rows(mem_rows, idx):
    mtot, d = mem_rows.shape
    btot = idx.shape[0]
    info = plsc.get_sparse_core_info()
    nw = info.num_cores * info.num_subcores
    b_per_w = btot // nw
    nchunks = 8
    ch = b_per_w // nchunks
    mesh = plsc.VectorSubcoreMesh(core_axis_name="c", subcore_axis_name="s")

    @functools.partial(
        pl.kernel,
        mesh=mesh,
        out_type=jax.ShapeDtypeStruct((btot, d), jnp.float32),
        scratch_types=[
            pltpu.VMEM((b_per_w,), jnp.int32),
            pltpu.VMEM((ch, d), jnp.float32),
            pltpu.VMEM((ch, d), jnp.float32),
            pltpu.SemaphoreType.DMA,
            pltpu.SemaphoreType.DMA,
        ],
    )
    def gather(mem_hbm, idx_hbm, out_hbm, idx_v, buf0, buf1, sem0, sem1):
        wid = lax.axis_index("s") * info.num_cores + lax.axis_index("c")
        base = wid * b_per_w
        pltpu.sync_copy(idx_hbm.at[pl.ds(base, b_per_w)], idx_v)
        bufs = (buf0, buf1)
        sems = (sem0, sem1)
        pending = [None] * nchunks
        for cidx in range(nchunks):
            pending[cidx] = pltpu.async_copy(
                mem_hbm.at[idx_v.at[pl.ds(cidx * ch, ch)]],
                bufs[cidx % 2], sems[cidx % 2])
            if cidx > 0:
                pending[cidx - 1].wait()
                pltpu.sync_copy(
                    bufs[(cidx - 1) % 2],
                    out_hbm.at[pl.ds(base + (cidx - 1) * ch, ch)])
        pending[nchunks - 1].wait()
        pltpu.sync_copy(
            bufs[(nchunks - 1) % 2],
            out_hbm.at[pl.ds(base + (nchunks - 1) * ch, ch)])

    return gather(mem_rows, idx)


def _combine_kernel(q_ref, y_ref, sel_ref, g_ref, wp_ref, o_ref,
                    *, scale, h, c):
    parts = []
    for hh in range(h):
        q = q_ref[hh]
        yh = y_ref[hh]
        logits = []
        for k in range(3):
            mk = sel_ref[k, :, pl.ds(hh * c, c)]
            logits.append(jnp.sum(q * mk, axis=1, keepdims=True) * scale)
        l0, l1, l2 = logits
        mx = jnp.maximum(jnp.maximum(l0, l1), l2)
        e0 = jnp.exp(l0 - mx)
        e1 = jnp.exp(l1 - mx)
        e2 = jnp.exp(l2 - mx)
        denom = e0 + e1 + e2
        ne_off = h * c
        mem_qkv = (e0 * sel_ref[0, :, pl.ds(ne_off + hh * c, c)]
                   + e1 * sel_ref[1, :, pl.ds(ne_off + hh * c, c)]
                   + e2 * sel_ref[2, :, pl.ds(ne_off + hh * c, c)]) / denom
        g = g_ref[hh, 0]
        parts.append(mem_qkv * g + yh * (1.0 - g))
    combined = jnp.concatenate(parts, axis=1)
    o_ref[...] = jnp.dot(combined, wp_ref[...],
                         preferred_element_type=jnp.float32)


def _combine_proj(qkv_h, y, sel, gate, w_proj, h, c, scale, tq=256):
    t = y.shape[1]
    ne = h * c
    kern = functools.partial(_combine_kernel, scale=scale, h=h, c=c)
    return pl.pallas_call(
        kern,
        grid=(t // tq,),
        in_specs=[
            pl.BlockSpec((h, tq, c), lambda i: (0, i, 0)),
            pl.BlockSpec((h, tq, c), lambda i: (0, i, 0)),
            pl.BlockSpec((3, tq, 2 * ne), lambda i: (0, i, 0)),
            pl.BlockSpec(memory_space=pltpu.SMEM),
            pl.BlockSpec((ne, ne), lambda i: (0, 0)),
        ],
        out_specs=pl.BlockSpec((tq, ne), lambda i: (i, 0)),
        out_shape=jax.ShapeDtypeStruct((t, ne), jnp.float32),
        compiler_params=pltpu.CompilerParams(
            dimension_semantics=("parallel",)),
    )(qkv_h, y, sel, gate, w_proj)


def kernel(x, mem_kv, W_attn, W_proj, gate_bias):
    b, t, h, c = x.shape
    ne = h * c
    m = mem_kv.shape[1]
    xf = x.reshape(t, ne)
    w_heads = W_attn.reshape(ne, 3 * h, c).transpose(1, 0, 2)

    qkv_h = _qkv_proj(xf, w_heads)
    y = _causal_attn(qkv_h, h)
    mem3 = mem_kv.reshape(m, 2, ne)
    topi = _knn_top3(qkv_h, mem3, h, c)
    idx = topi[:3].reshape(3 * t)
    mem_rows = mem_kv.reshape(m, 2 * ne)
    sel = _gather_rows(mem_rows, idx)
    sel = sel.reshape(3, t, 2 * ne)
    scale = ne / (h ** (-0.5))
    gate = gate_bias.reshape(h, 1)
    out = _combine_proj(qkv_h, y, sel, gate, W_proj, h, c, scale)
    return out.reshape(b, t, ne)


# speedup vs baseline: 40.2600x; 1.0209x over previous
import functools

import jax
import jax.numpy as jnp
from jax import lax
from jax.experimental import pallas as pl
from jax.experimental.pallas import tpu as pltpu
from jax.experimental.pallas import tpu_sc as plsc


def _qkv_kernel(x_ref, w_ref, o_ref):
    o_ref[0] = jnp.dot(x_ref[...], w_ref[0],
                       preferred_element_type=jnp.float32)


def _qkv_proj(xf, w_heads):
    t, ne = xf.shape
    nh3 = w_heads.shape[0]
    c = w_heads.shape[2]
    return pl.pallas_call(
        _qkv_kernel,
        grid=(nh3,),
        in_specs=[
            pl.BlockSpec((t, ne), lambda j: (0, 0)),
            pl.BlockSpec((1, ne, c), lambda j: (j, 0, 0)),
        ],
        out_specs=pl.BlockSpec((1, t, c), lambda j: (j, 0, 0)),
        out_shape=jax.ShapeDtypeStruct((nh3, t, c), jnp.float32),
        compiler_params=pltpu.CompilerParams(
            dimension_semantics=("parallel",)),
    )(xf, w_heads)


def _causal_kernel(q_ref, k_ref, v_ref, o_ref, *, tq, c):
    i = pl.program_id(1)
    q = q_ref[0]
    k = k_ref[0]
    v = v_ref[0]
    s = lax.dot_general(q, k, (((1,), (1,)), ((), ())),
                        preferred_element_type=jnp.float32)
    s = s * (1.0 / (c ** 0.5))
    row = lax.broadcasted_iota(jnp.int32, s.shape, 0) + i * tq
    col = lax.broadcasted_iota(jnp.int32, s.shape, 1)
    s = jnp.where(col <= row, s, -1e30)
    m = jnp.max(s, axis=1, keepdims=True)
    p = jnp.exp(s - m)
    denom = jnp.sum(p, axis=1, keepdims=True)
    y = lax.dot_general(p, v, (((1,), (0,)), ((), ())),
                        preferred_element_type=jnp.float32)
    o_ref[0] = y / denom


def _causal_attn(qkv_h, h, tq=512):
    t = qkv_h.shape[1]
    c = qkv_h.shape[2]
    kern = functools.partial(_causal_kernel, tq=tq, c=c)
    return pl.pallas_call(
        kern,
        grid=(h, t // tq),
        in_specs=[
            pl.BlockSpec((1, tq, c), lambda hh, i: (hh, i, 0)),
            pl.BlockSpec((1, t, c), lambda hh, i: (h + hh, 0, 0)),
            pl.BlockSpec((1, t, c), lambda hh, i: (2 * h + hh, 0, 0)),
        ],
        out_specs=pl.BlockSpec((1, tq, c), lambda hh, i: (hh, i, 0)),
        out_shape=jax.ShapeDtypeStruct((h, t, c), jnp.float32),
        compiler_params=pltpu.CompilerParams(
            dimension_semantics=("parallel", "parallel")),
    )(qkv_h, qkv_h, qkv_h)


def _knn_kernel(q_ref, memk_ref, topi_ref,
                v0, v1, v2, i0, i1, i2, *, mt, t, h, c):
    step = pl.program_id(0)
    nsteps = pl.num_programs(0)

    @pl.when(step == 0)
    def _init():
        neg = jnp.full((1, t), -jnp.inf, jnp.float32)
        zero = jnp.zeros((1, t), jnp.int32)
        v0[...] = neg
        v1[...] = neg
        v2[...] = neg
        i0[...] = zero
        i1[...] = zero
        i2[...] = zero

    s = jnp.zeros((mt, t), jnp.float32)
    for hh in range(h):
        k_h = memk_ref[:, 0, pl.ds(hh * c, c)]
        q_h = q_ref[hh]
        s = s + lax.dot_general(k_h, q_h, (((1,), (1,)), ((), ())),
                                preferred_element_type=jnp.float32)
    rows = lax.broadcasted_iota(jnp.int32, (mt, t), 0)
    base = step * mt
    for _ in range(3):
        m = jnp.max(s, axis=0, keepdims=True)
        cand = jnp.where(s == m, rows, mt)
        am = jnp.min(cand, axis=0, keepdims=True)
        s = jnp.where(rows == am, -jnp.inf, s)
        ci = am + base
        c0, c1, c2 = v0[...], v1[...], v2[...]
        j0, j1, j2 = i0[...], i1[...], i2[...]
        gt0 = m > c0
        gt1 = m > c1
        gt2 = m > c2
        v0[...] = jnp.where(gt0, m, c0)
        i0[...] = jnp.where(gt0, ci, j0)
        v1[...] = jnp.where(gt0, c0, jnp.where(gt1, m, c1))
        i1[...] = jnp.where(gt0, j0, jnp.where(gt1, ci, j1))
        v2[...] = jnp.where(gt1, c1, jnp.where(gt2, m, c2))
        i2[...] = jnp.where(gt1, j1, jnp.where(gt2, ci, j2))

    @pl.when(step == nsteps - 1)
    def _emit():
        topi_ref[...] = jnp.zeros((8, t), jnp.int32)
        topi_ref[0:1, :] = i0[...]
        topi_ref[1:2, :] = i1[...]
        topi_ref[2:3, :] = i2[...]


def _knn_top3(qkv_h, mem3, h, c, mt=1024):
    t = qkv_h.shape[1]
    ne = h * c
    m = mem3.shape[0]
    kern = functools.partial(_knn_kernel, mt=mt, t=t, h=h, c=c)
    return pl.pallas_call(
        kern,
        grid=(m // mt,),
        in_specs=[
            pl.BlockSpec((h, t, c), lambda i: (0, 0, 0)),
            pl.BlockSpec((mt, 2, ne), lambda i: (i, 0, 0)),
        ],
        out_specs=pl.BlockSpec((8, t), lambda i: (0, 0)),
        out_shape=jax.ShapeDtypeStruct((8, t), jnp.int32),
        scratch_shapes=[pltpu.VMEM((1, t), jnp.float32)] * 3
        + [pltpu.VMEM((1, t), jnp.int32)] * 3,
        compiler_params=pltpu.CompilerParams(
            dimension_semantics=("arbitrary",)),
    )(qkv_h, mem3)


def _gather_rows(mem3, idx):
    mtot, two, d = mem3.shape
    btot = idx.shape[0]
    info = plsc.get_sparse_core_info()
    nw = info.num_cores * info.num_subcores
    b_per_w = btot // nw
    nchunks = 8
    ch = b_per_w // nchunks
    mesh = plsc.VectorSubcoreMesh(core_axis_name="c", subcore_axis_name="s")

    @functools.partial(
        pl.kernel,
        mesh=mesh,
        out_type=jax.ShapeDtypeStruct((btot, two, d), jnp.float32),
        scratch_types=[
            pltpu.VMEM((b_per_w,), jnp.int32),
            pltpu.VMEM((ch, two, d), jnp.float32),
            pltpu.VMEM((ch, two, d), jnp.float32),
            pltpu.SemaphoreType.DMA,
            pltpu.SemaphoreType.DMA,
        ],
    )
    def gather(mem_hbm, idx_hbm, out_hbm, idx_v, buf0, buf1, sem0, sem1):
        wid = lax.axis_index("s") * info.num_cores + lax.axis_index("c")
        base = wid * b_per_w
        pltpu.sync_copy(idx_hbm.at[pl.ds(base, b_per_w)], idx_v)
        bufs = (buf0, buf1)
        sems = (sem0, sem1)
        pending = [None] * nchunks
        for cidx in range(nchunks):
            pending[cidx] = pltpu.async_copy(
                mem_hbm.at[idx_v.at[pl.ds(cidx * ch, ch)]],
                bufs[cidx % 2], sems[cidx % 2])
            if cidx > 0:
                pending[cidx - 1].wait()
                pltpu.sync_copy(
                    bufs[(cidx - 1) % 2],
                    out_hbm.at[pl.ds(base + (cidx - 1) * ch, ch)])
        pending[nchunks - 1].wait()
        pltpu.sync_copy(
            bufs[(nchunks - 1) % 2],
            out_hbm.at[pl.ds(base + (nchunks - 1) * ch, ch)])

    return gather(mem3, idx)


def _combine_kernel(q_ref, y_ref, sel_ref, g_ref, wp_ref, o_ref,
                    *, scale, h, c):
    parts = []
    for hh in range(h):
        q = q_ref[hh]
        yh = y_ref[hh]
        logits = []
        for k in range(3):
            mk = sel_ref[k, :, 0, pl.ds(hh * c, c)]
            logits.append(jnp.sum(q * mk, axis=1, keepdims=True) * scale)
        l0, l1, l2 = logits
        mx = jnp.maximum(jnp.maximum(l0, l1), l2)
        e0 = jnp.exp(l0 - mx)
        e1 = jnp.exp(l1 - mx)
        e2 = jnp.exp(l2 - mx)
        denom = e0 + e1 + e2
        mem_qkv = (e0 * sel_ref[0, :, 1, pl.ds(hh * c, c)]
                   + e1 * sel_ref[1, :, 1, pl.ds(hh * c, c)]
                   + e2 * sel_ref[2, :, 1, pl.ds(hh * c, c)]) / denom
        g = g_ref[hh, 0]
        parts.append(mem_qkv * g + yh * (1.0 - g))
    combined = jnp.concatenate(parts, axis=1)
    o_ref[...] = jnp.dot(combined, wp_ref[...],
                         preferred_element_type=jnp.float32)


def _combine_proj(qkv_h, y, sel, gate, w_proj, h, c, scale, tq=256):
    t = y.shape[1]
    ne = h * c
    kern = functools.partial(_combine_kernel, scale=scale, h=h, c=c)
    return pl.pallas_call(
        kern,
        grid=(t // tq,),
        in_specs=[
            pl.BlockSpec((h, tq, c), lambda i: (0, i, 0)),
            pl.BlockSpec((h, tq, c), lambda i: (0, i, 0)),
            pl.BlockSpec((3, tq, 2, ne), lambda i: (0, i, 0, 0)),
            pl.BlockSpec(memory_space=pltpu.SMEM),
            pl.BlockSpec((ne, ne), lambda i: (0, 0)),
        ],
        out_specs=pl.BlockSpec((tq, ne), lambda i: (i, 0)),
        out_shape=jax.ShapeDtypeStruct((t, ne), jnp.float32),
        compiler_params=pltpu.CompilerParams(
            dimension_semantics=("parallel",)),
    )(qkv_h, y, sel, gate, w_proj)


def kernel(x, mem_kv, W_attn, W_proj, gate_bias):
    b, t, h, c = x.shape
    ne = h * c
    m = mem_kv.shape[1]
    xf = x.reshape(t, ne)
    w_heads = W_attn.reshape(ne, 3 * h, c).transpose(1, 0, 2)

    qkv_h = _qkv_proj(xf, w_heads)
    y = _causal_attn(qkv_h, h)
    mem3 = mem_kv.reshape(m, 2, ne)
    topi = _knn_top3(qkv_h, mem3, h, c)
    idx = topi[:3].reshape(3 * t)
    sel = _gather_rows(mem3, idx)
    sel = sel.reshape(3, t, 2, ne)
    scale = ne / (h ** (-0.5))
    gate = gate_bias.reshape(h, 1)
    out = _combine_proj(qkv_h, y, sel, gate, W_proj, h, c, scale)
    return out.reshape(b, t, ne)


# speedup vs baseline: 62.7453x; 1.5585x over previous
import functools

import jax
import jax.numpy as jnp
from jax import lax
from jax.experimental import pallas as pl
from jax.experimental.pallas import tpu as pltpu
from jax.experimental.pallas import tpu_sc as plsc


def _qkv_kernel(x_ref, w_ref, o_ref):
    o_ref[0] = jnp.dot(x_ref[...], w_ref[0],
                       preferred_element_type=jnp.float32)


def _qkv_proj(xf, w_heads):
    t, ne = xf.shape
    nh3 = w_heads.shape[0]
    c = w_heads.shape[2]
    return pl.pallas_call(
        _qkv_kernel,
        grid=(nh3,),
        in_specs=[
            pl.BlockSpec((t, ne), lambda j: (0, 0)),
            pl.BlockSpec((1, ne, c), lambda j: (j, 0, 0)),
        ],
        out_specs=pl.BlockSpec((1, t, c), lambda j: (j, 0, 0)),
        out_shape=jax.ShapeDtypeStruct((nh3, t, c), jnp.float32),
        compiler_params=pltpu.CompilerParams(
            dimension_semantics=("parallel",)),
    )(xf, w_heads)


def _qflat_kernel(x_ref, w_ref, o_ref):
    o_ref[...] = jnp.dot(x_ref[...], w_ref[...],
                         preferred_element_type=jnp.float32)


def _q_flat(xf, w_attn):
    t, ne = xf.shape
    return pl.pallas_call(
        _qflat_kernel,
        grid=(1,),
        in_specs=[
            pl.BlockSpec((t, ne), lambda i: (0, 0)),
            pl.BlockSpec((ne, ne), lambda i: (0, 0)),
        ],
        out_specs=pl.BlockSpec((t, ne), lambda i: (0, 0)),
        out_shape=jax.ShapeDtypeStruct((t, ne), jnp.float32),
    )(xf, w_attn)


def _causal_kernel(q_ref, k_ref, v_ref, o_ref, *, tq, c):
    i = pl.program_id(1)
    q = q_ref[0]
    k = k_ref[0]
    v = v_ref[0]
    s = lax.dot_general(q, k, (((1,), (1,)), ((), ())),
                        preferred_element_type=jnp.float32)
    s = s * (1.0 / (c ** 0.5))
    row = lax.broadcasted_iota(jnp.int32, s.shape, 0) + i * tq
    col = lax.broadcasted_iota(jnp.int32, s.shape, 1)
    s = jnp.where(col <= row, s, -1e30)
    m = jnp.max(s, axis=1, keepdims=True)
    p = jnp.exp(s - m)
    denom = jnp.sum(p, axis=1, keepdims=True)
    y = lax.dot_general(p, v, (((1,), (0,)), ((), ())),
                        preferred_element_type=jnp.float32)
    o_ref[0] = y / denom


def _causal_attn(qkv_h, h, tq=512):
    t = qkv_h.shape[1]
    c = qkv_h.shape[2]
    kern = functools.partial(_causal_kernel, tq=tq, c=c)
    return pl.pallas_call(
        kern,
        grid=(h, t // tq),
        in_specs=[
            pl.BlockSpec((1, tq, c), lambda hh, i: (hh, i, 0)),
            pl.BlockSpec((1, t, c), lambda hh, i: (h + hh, 0, 0)),
            pl.BlockSpec((1, t, c), lambda hh, i: (2 * h + hh, 0, 0)),
        ],
        out_specs=pl.BlockSpec((1, tq, c), lambda hh, i: (hh, i, 0)),
        out_shape=jax.ShapeDtypeStruct((h, t, c), jnp.float32),
        compiler_params=pltpu.CompilerParams(
            dimension_semantics=("parallel", "parallel")),
    )(qkv_h, qkv_h, qkv_h)


def _knn_kernel(q_ref, memk_ref, topi_ref,
                v0, v1, v2, i0, i1, i2, *, mt, t):
    step = pl.program_id(0)
    nsteps = pl.num_programs(0)

    @pl.when(step == 0)
    def _init():
        neg = jnp.full((1, t), -jnp.inf, jnp.float32)
        zero = jnp.zeros((1, t), jnp.int32)
        v0[...] = neg
        v1[...] = neg
        v2[...] = neg
        i0[...] = zero
        i1[...] = zero
        i2[...] = zero

    k_tile = memk_ref[:, 0, :]
    s = lax.dot_general(k_tile, q_ref[...], (((1,), (1,)), ((), ())),
                        preferred_element_type=jnp.float32)
    rows = lax.broadcasted_iota(jnp.int32, (mt, t), 0)
    base = step * mt
    prev = []
    for _ in range(3):
        if not prev:
            m = jnp.max(s, axis=0, keepdims=True)
            cand = jnp.where(s == m, rows, mt)
        else:
            excl = rows == prev[0]
            for p in prev[1:]:
                excl = excl | (rows == p)
            m = jnp.max(jnp.where(excl, -jnp.inf, s),
                        axis=0, keepdims=True)
            cand = jnp.where((s == m) & ~excl, rows, mt)
        am = jnp.min(cand, axis=0, keepdims=True)
        prev.append(am)
        ci = am + base
        c0, c1, c2 = v0[...], v1[...], v2[...]
        j0, j1, j2 = i0[...], i1[...], i2[...]
        gt0 = m > c0
        gt1 = m > c1
        gt2 = m > c2
        v0[...] = jnp.where(gt0, m, c0)
        i0[...] = jnp.where(gt0, ci, j0)
        v1[...] = jnp.where(gt0, c0, jnp.where(gt1, m, c1))
        i1[...] = jnp.where(gt0, j0, jnp.where(gt1, ci, j1))
        v2[...] = jnp.where(gt1, c1, jnp.where(gt2, m, c2))
        i2[...] = jnp.where(gt1, j1, jnp.where(gt2, ci, j2))

    @pl.when(step == nsteps - 1)
    def _emit():
        topi_ref[...] = jnp.zeros((8, t), jnp.int32)
        topi_ref[0:1, :] = i0[...]
        topi_ref[1:2, :] = i1[...]
        topi_ref[2:3, :] = i2[...]


def _knn_top3(q_flat, mem3, mt=1024):
    t, ne = q_flat.shape
    m = mem3.shape[0]
    kern = functools.partial(_knn_kernel, mt=mt, t=t)
    return pl.pallas_call(
        kern,
        grid=(m // mt,),
        in_specs=[
            pl.BlockSpec((t, ne), lambda i: (0, 0)),
            pl.BlockSpec((mt, 2, ne), lambda i: (i, 0, 0)),
        ],
        out_specs=pl.BlockSpec((8, t), lambda i: (0, 0)),
        out_shape=jax.ShapeDtypeStruct((8, t), jnp.int32),
        scratch_shapes=[pltpu.VMEM((1, t), jnp.float32)] * 3
        + [pltpu.VMEM((1, t), jnp.int32)] * 3,
        compiler_params=pltpu.CompilerParams(
            dimension_semantics=("arbitrary",)),
    )(q_flat, mem3)


def _gather_rows(mem3, idx):
    mtot, two, d = mem3.shape
    btot = idx.shape[0]
    info = plsc.get_sparse_core_info()
    nw = info.num_cores * info.num_subcores
    b_per_w = btot // nw
    nchunks = 8
    ch = b_per_w // nchunks
    mesh = plsc.VectorSubcoreMesh(core_axis_name="c", subcore_axis_name="s")

    @functools.partial(
        pl.kernel,
        mesh=mesh,
        out_type=jax.ShapeDtypeStruct((btot, two, d), jnp.float32),
        scratch_types=[
            pltpu.VMEM((b_per_w,), jnp.int32),
            pltpu.VMEM((ch, two, d), jnp.float32),
            pltpu.VMEM((ch, two, d), jnp.float32),
            pltpu.SemaphoreType.DMA,
            pltpu.SemaphoreType.DMA,
        ],
    )
    def gather(mem_hbm, idx_hbm, out_hbm, idx_v, buf0, buf1, sem0, sem1):
        wid = lax.axis_index("s") * info.num_cores + lax.axis_index("c")
        base = wid * b_per_w
        pltpu.sync_copy(idx_hbm.at[pl.ds(base, b_per_w)], idx_v)
        bufs = (buf0, buf1)
        sems = (sem0, sem1)
        pending = [None] * nchunks
        for cidx in range(nchunks):
            pending[cidx] = pltpu.async_copy(
                mem_hbm.at[idx_v.at[pl.ds(cidx * ch, ch)]],
                bufs[cidx % 2], sems[cidx % 2])
            if cidx > 0:
                pending[cidx - 1].wait()
                pltpu.sync_copy(
                    bufs[(cidx - 1) % 2],
                    out_hbm.at[pl.ds(base + (cidx - 1) * ch, ch)])
        pending[nchunks - 1].wait()
        pltpu.sync_copy(
            bufs[(nchunks - 1) % 2],
            out_hbm.at[pl.ds(base + (nchunks - 1) * ch, ch)])

    return gather(mem3, idx)


def _combine_kernel(q_ref, y_ref, sel_ref, g_ref, wp_ref, o_ref,
                    *, scale, h, c):
    parts = []
    for hh in range(h):
        q = q_ref[hh]
        yh = y_ref[hh]
        logits = []
        for k in range(3):
            mk = sel_ref[k, :, 0, pl.ds(hh * c, c)]
            logits.append(jnp.sum(q * mk, axis=1, keepdims=True) * scale)
        l0, l1, l2 = logits
        mx = jnp.maximum(jnp.maximum(l0, l1), l2)
        e0 = jnp.exp(l0 - mx)
        e1 = jnp.exp(l1 - mx)
        e2 = jnp.exp(l2 - mx)
        denom = e0 + e1 + e2
        mem_qkv = (e0 * sel_ref[0, :, 1, pl.ds(hh * c, c)]
                   + e1 * sel_ref[1, :, 1, pl.ds(hh * c, c)]
                   + e2 * sel_ref[2, :, 1, pl.ds(hh * c, c)]) / denom
        g = g_ref[hh, 0]
        parts.append(mem_qkv * g + yh * (1.0 - g))
    combined = jnp.concatenate(parts, axis=1)
    o_ref[...] = jnp.dot(combined, wp_ref[...],
                         preferred_element_type=jnp.float32)


def _combine_proj(qkv_h, y, sel, gate, w_proj, h, c, scale, tq=256):
    t = y.shape[1]
    ne = h * c
    kern = functools.partial(_combine_kernel, scale=scale, h=h, c=c)
    return pl.pallas_call(
        kern,
        grid=(t // tq,),
        in_specs=[
            pl.BlockSpec((h, tq, c), lambda i: (0, i, 0)),
            pl.BlockSpec((h, tq, c), lambda i: (0, i, 0)),
            pl.BlockSpec((3, tq, 2, ne), lambda i: (0, i, 0, 0)),
            pl.BlockSpec(memory_space=pltpu.SMEM),
            pl.BlockSpec((ne, ne), lambda i: (0, 0)),
        ],
        out_specs=pl.BlockSpec((tq, ne), lambda i: (i, 0)),
        out_shape=jax.ShapeDtypeStruct((t, ne), jnp.float32),
        compiler_params=pltpu.CompilerParams(
            dimension_semantics=("parallel",)),
    )(qkv_h, y, sel, gate, w_proj)


def kernel(x, mem_kv, W_attn, W_proj, gate_bias):
    b, t, h, c = x.shape
    ne = h * c
    m = mem_kv.shape[1]
    xf = x.reshape(t, ne)
    w_heads = W_attn.reshape(ne, 3 * h, c).transpose(1, 0, 2)

    qkv_h = _qkv_proj(xf, w_heads)
    y = _causal_attn(qkv_h, h)
    q_flat = _q_flat(xf, W_attn)
    mem3 = mem_kv.reshape(m, 2, ne)
    topi = _knn_top3(q_flat, mem3)
    idx = topi[:3].reshape(3 * t)
    sel = _gather_rows(mem3, idx)
    sel = sel.reshape(3, t, 2, ne)
    scale = ne / (h ** (-0.5))
    gate = gate_bias.reshape(h, 1)
    out = _combine_proj(qkv_h, y, sel, gate, W_proj, h, c, scale)
    return out.reshape(b, t, ne)


# speedup vs baseline: 72.1183x; 1.1494x over previous
import functools

import jax
import jax.numpy as jnp
from jax import lax
from jax.experimental import pallas as pl
from jax.experimental.pallas import tpu as pltpu
from jax.experimental.pallas import tpu_sc as plsc


def _qkv_kernel(x_ref, w_ref, o_ref):
    o_ref[0] = jnp.dot(x_ref[...], w_ref[0],
                       preferred_element_type=jnp.float32)


def _qkv_proj(xf, w_heads):
    t, ne = xf.shape
    nh3 = w_heads.shape[0]
    c = w_heads.shape[2]
    return pl.pallas_call(
        _qkv_kernel,
        grid=(nh3,),
        in_specs=[
            pl.BlockSpec((t, ne), lambda j: (0, 0)),
            pl.BlockSpec((1, ne, c), lambda j: (j, 0, 0)),
        ],
        out_specs=pl.BlockSpec((1, t, c), lambda j: (j, 0, 0)),
        out_shape=jax.ShapeDtypeStruct((nh3, t, c), jnp.float32),
        compiler_params=pltpu.CompilerParams(
            dimension_semantics=("parallel",)),
    )(xf, w_heads)


def _qflat_kernel(x_ref, w_ref, o_ref):
    o_ref[...] = jnp.dot(x_ref[...], w_ref[...],
                         preferred_element_type=jnp.float32)


def _q_flat(xf, w_attn):
    t, ne = xf.shape
    return pl.pallas_call(
        _qflat_kernel,
        grid=(1,),
        in_specs=[
            pl.BlockSpec((t, ne), lambda i: (0, 0)),
            pl.BlockSpec((ne, ne), lambda i: (0, 0)),
        ],
        out_specs=pl.BlockSpec((t, ne), lambda i: (0, 0)),
        out_shape=jax.ShapeDtypeStruct((t, ne), jnp.float32),
    )(xf, w_attn)


def _causal_kernel(q_ref, k_ref, v_ref, o_ref, *, tq, c):
    i = pl.program_id(1)
    q = q_ref[0]
    k = k_ref[0]
    v = v_ref[0]
    s = lax.dot_general(q, k, (((1,), (1,)), ((), ())),
                        preferred_element_type=jnp.float32)
    s = s * (1.0 / (c ** 0.5))
    row = lax.broadcasted_iota(jnp.int32, s.shape, 0) + i * tq
    col = lax.broadcasted_iota(jnp.int32, s.shape, 1)
    s = jnp.where(col <= row, s, -1e30)
    m = jnp.max(s, axis=1, keepdims=True)
    p = jnp.exp(s - m)
    denom = jnp.sum(p, axis=1, keepdims=True)
    y = lax.dot_general(p, v, (((1,), (0,)), ((), ())),
                        preferred_element_type=jnp.float32)
    o_ref[0] = y / denom


def _causal_attn(qkv_h, h, tq=512):
    t = qkv_h.shape[1]
    c = qkv_h.shape[2]
    kern = functools.partial(_causal_kernel, tq=tq, c=c)
    return pl.pallas_call(
        kern,
        grid=(h, t // tq),
        in_specs=[
            pl.BlockSpec((1, tq, c), lambda hh, i: (hh, i, 0)),
            pl.BlockSpec((1, t, c), lambda hh, i: (h + hh, 0, 0)),
            pl.BlockSpec((1, t, c), lambda hh, i: (2 * h + hh, 0, 0)),
        ],
        out_specs=pl.BlockSpec((1, tq, c), lambda hh, i: (hh, i, 0)),
        out_shape=jax.ShapeDtypeStruct((h, t, c), jnp.float32),
        compiler_params=pltpu.CompilerParams(
            dimension_semantics=("parallel", "parallel")),
    )(qkv_h, qkv_h, qkv_h)


def _knn_kernel(q_ref, memk_ref, topi_ref,
                v0, v1, v2, i0, i1, i2, *, mt, t):
    step = pl.program_id(0)
    nsteps = pl.num_programs(0)

    @pl.when(step == 0)
    def _init():
        neg = jnp.full((1, t), -jnp.inf, jnp.float32)
        zero = jnp.zeros((1, t), jnp.int32)
        v0[...] = neg
        v1[...] = neg
        v2[...] = neg
        i0[...] = zero
        i1[...] = zero
        i2[...] = zero

    k_tile = memk_ref[:, 0, :]
    s = lax.dot_general(k_tile, q_ref[...], (((1,), (1,)), ((), ())),
                        preferred_element_type=jnp.float32)
    rows = lax.broadcasted_iota(jnp.int32, (mt, t), 0)
    base = step * mt
    am = None
    for _ in range(3):
        if am is not None:
            s = jnp.where(rows == am, -jnp.inf, s)
        m = jnp.max(s, axis=0, keepdims=True)
        cand = jnp.where(s == m, rows, mt)
        am = jnp.min(cand, axis=0, keepdims=True)
        ci = am + base
        c0, c1, c2 = v0[...], v1[...], v2[...]
        j0, j1, j2 = i0[...], i1[...], i2[...]
        gt0 = m > c0
        gt1 = m > c1
        gt2 = m > c2
        v0[...] = jnp.where(gt0, m, c0)
        i0[...] = jnp.where(gt0, ci, j0)
        v1[...] = jnp.where(gt0, c0, jnp.where(gt1, m, c1))
        i1[...] = jnp.where(gt0, j0, jnp.where(gt1, ci, j1))
        v2[...] = jnp.where(gt1, c1, jnp.where(gt2, m, c2))
        i2[...] = jnp.where(gt1, j1, jnp.where(gt2, ci, j2))

    @pl.when(step == nsteps - 1)
    def _emit():
        topi_ref[...] = jnp.zeros((8, t), jnp.int32)
        topi_ref[0:1, :] = i0[...]
        topi_ref[1:2, :] = i1[...]
        topi_ref[2:3, :] = i2[...]


def _knn_top3(q_flat, mem3, mt=1024):
    t, ne = q_flat.shape
    m = mem3.shape[0]
    kern = functools.partial(_knn_kernel, mt=mt, t=t)
    return pl.pallas_call(
        kern,
        grid=(m // mt,),
        in_specs=[
            pl.BlockSpec((t, ne), lambda i: (0, 0)),
            pl.BlockSpec((mt, 2, ne), lambda i: (i, 0, 0)),
        ],
        out_specs=pl.BlockSpec((8, t), lambda i: (0, 0)),
        out_shape=jax.ShapeDtypeStruct((8, t), jnp.int32),
        scratch_shapes=[pltpu.VMEM((1, t), jnp.float32)] * 3
        + [pltpu.VMEM((1, t), jnp.int32)] * 3,
        compiler_params=pltpu.CompilerParams(
            dimension_semantics=("arbitrary",)),
    )(q_flat, mem3)


def _gather_rows(mem3, idx):
    mtot, two, d = mem3.shape
    btot = idx.shape[0]
    info = plsc.get_sparse_core_info()
    nw = info.num_cores * info.num_subcores
    b_per_w = btot // nw
    nchunks = 8
    ch = b_per_w // nchunks
    mesh = plsc.VectorSubcoreMesh(core_axis_name="c", subcore_axis_name="s")

    @functools.partial(
        pl.kernel,
        mesh=mesh,
        out_type=jax.ShapeDtypeStruct((btot, two, d), jnp.float32),
        scratch_types=[
            pltpu.VMEM((b_per_w,), jnp.int32),
            pltpu.VMEM((ch, two, d), jnp.float32),
            pltpu.VMEM((ch, two, d), jnp.float32),
            pltpu.SemaphoreType.DMA,
            pltpu.SemaphoreType.DMA,
        ],
    )
    def gather(mem_hbm, idx_hbm, out_hbm, idx_v, buf0, buf1, sem0, sem1):
        wid = lax.axis_index("s") * info.num_cores + lax.axis_index("c")
        base = wid * b_per_w
        pltpu.sync_copy(idx_hbm.at[pl.ds(base, b_per_w)], idx_v)
        bufs = (buf0, buf1)
        sems = (sem0, sem1)
        pending = [None] * nchunks
        for cidx in range(nchunks):
            pending[cidx] = pltpu.async_copy(
                mem_hbm.at[idx_v.at[pl.ds(cidx * ch, ch)]],
                bufs[cidx % 2], sems[cidx % 2])
            if cidx > 0:
                pending[cidx - 1].wait()
                pltpu.sync_copy(
                    bufs[(cidx - 1) % 2],
                    out_hbm.at[pl.ds(base + (cidx - 1) * ch, ch)])
        pending[nchunks - 1].wait()
        pltpu.sync_copy(
            bufs[(nchunks - 1) % 2],
            out_hbm.at[pl.ds(base + (nchunks - 1) * ch, ch)])

    return gather(mem3, idx)


def _combine_kernel(q_ref, y_ref, sel_ref, g_ref, wp_ref, o_ref,
                    *, scale, h, c):
    parts = []
    for hh in range(h):
        q = q_ref[hh]
        yh = y_ref[hh]
        logits = []
        for k in range(3):
            mk = sel_ref[k, :, 0, pl.ds(hh * c, c)]
            logits.append(jnp.sum(q * mk, axis=1, keepdims=True) * scale)
        l0, l1, l2 = logits
        mx = jnp.maximum(jnp.maximum(l0, l1), l2)
        e0 = jnp.exp(l0 - mx)
        e1 = jnp.exp(l1 - mx)
        e2 = jnp.exp(l2 - mx)
        denom = e0 + e1 + e2
        mem_qkv = (e0 * sel_ref[0, :, 1, pl.ds(hh * c, c)]
                   + e1 * sel_ref[1, :, 1, pl.ds(hh * c, c)]
                   + e2 * sel_ref[2, :, 1, pl.ds(hh * c, c)]) / denom
        g = g_ref[hh, 0]
        parts.append(mem_qkv * g + yh * (1.0 - g))
    combined = jnp.concatenate(parts, axis=1)
    o_ref[...] = jnp.dot(combined, wp_ref[...],
                         preferred_element_type=jnp.float32)


def _combine_proj(qkv_h, y, sel, gate, w_proj, h, c, scale, tq=256):
    t = y.shape[1]
    ne = h * c
    kern = functools.partial(_combine_kernel, scale=scale, h=h, c=c)
    return pl.pallas_call(
        kern,
        grid=(t // tq,),
        in_specs=[
            pl.BlockSpec((h, tq, c), lambda i: (0, i, 0)),
            pl.BlockSpec((h, tq, c), lambda i: (0, i, 0)),
            pl.BlockSpec((3, tq, 2, ne), lambda i: (0, i, 0, 0)),
            pl.BlockSpec(memory_space=pltpu.SMEM),
            pl.BlockSpec((ne, ne), lambda i: (0, 0)),
        ],
        out_specs=pl.BlockSpec((tq, ne), lambda i: (i, 0)),
        out_shape=jax.ShapeDtypeStruct((t, ne), jnp.float32),
        compiler_params=pltpu.CompilerParams(
            dimension_semantics=("parallel",)),
    )(qkv_h, y, sel, gate, w_proj)


def kernel(x, mem_kv, W_attn, W_proj, gate_bias):
    b, t, h, c = x.shape
    ne = h * c
    m = mem_kv.shape[1]
    xf = x.reshape(t, ne)
    w_heads = W_attn.reshape(ne, 3 * h, c).transpose(1, 0, 2)

    qkv_h = _qkv_proj(xf, w_heads)
    y = _causal_attn(qkv_h, h)
    q_flat = _q_flat(xf, W_attn)
    mem3 = mem_kv.reshape(m, 2, ne)
    topi = _knn_top3(q_flat, mem3)
    idx = topi[:3].reshape(3 * t)
    sel = _gather_rows(mem3, idx)
    sel = sel.reshape(3, t, 2, ne)
    scale = ne / (h ** (-0.5))
    gate = gate_bias.reshape(h, 1)
    out = _combine_proj(qkv_h, y, sel, gate, W_proj, h, c, scale)
    return out.reshape(b, t, ne)


# speedup vs baseline: 73.3302x; 1.0168x over previous
import functools

import jax
import jax.numpy as jnp
from jax import lax
from jax.experimental import pallas as pl
from jax.experimental.pallas import tpu as pltpu
from jax.experimental.pallas import tpu_sc as plsc


def _qkv_kernel(x_ref, w_ref, o_ref):
    o_ref[0] = jnp.dot(x_ref[...], w_ref[0],
                       preferred_element_type=jnp.float32)


def _qkv_proj(xf, w_heads):
    t, ne = xf.shape
    nh3 = w_heads.shape[0]
    c = w_heads.shape[2]
    return pl.pallas_call(
        _qkv_kernel,
        grid=(nh3,),
        in_specs=[
            pl.BlockSpec((t, ne), lambda j: (0, 0)),
            pl.BlockSpec((1, ne, c), lambda j: (j, 0, 0)),
        ],
        out_specs=pl.BlockSpec((1, t, c), lambda j: (j, 0, 0)),
        out_shape=jax.ShapeDtypeStruct((nh3, t, c), jnp.float32),
        compiler_params=pltpu.CompilerParams(
            dimension_semantics=("parallel",)),
    )(xf, w_heads)


def _qflat_kernel(x_ref, w_ref, o_ref):
    o_ref[...] = jnp.dot(x_ref[...], w_ref[...],
                         preferred_element_type=jnp.float32)


def _q_flat(xf, w_attn):
    t, ne = xf.shape
    return pl.pallas_call(
        _qflat_kernel,
        grid=(1,),
        in_specs=[
            pl.BlockSpec((t, ne), lambda i: (0, 0)),
            pl.BlockSpec((ne, ne), lambda i: (0, 0)),
        ],
        out_specs=pl.BlockSpec((t, ne), lambda i: (0, 0)),
        out_shape=jax.ShapeDtypeStruct((t, ne), jnp.float32),
    )(xf, w_attn)


def _causal_kernel(q_ref, k_ref, v_ref, o_ref, *, tq, c):
    i = pl.program_id(1)
    q = q_ref[0]
    k = k_ref[0]
    v = v_ref[0]
    s = lax.dot_general(q, k, (((1,), (1,)), ((), ())),
                        preferred_element_type=jnp.float32)
    s = s * (1.0 / (c ** 0.5))
    row = lax.broadcasted_iota(jnp.int32, s.shape, 0) + i * tq
    col = lax.broadcasted_iota(jnp.int32, s.shape, 1)
    s = jnp.where(col <= row, s, -1e30)
    m = jnp.max(s, axis=1, keepdims=True)
    p = jnp.exp(s - m)
    denom = jnp.sum(p, axis=1, keepdims=True)
    y = lax.dot_general(p, v, (((1,), (0,)), ((), ())),
                        preferred_element_type=jnp.float32)
    o_ref[0] = y / denom


def _causal_attn(qkv_h, h, tq=512):
    t = qkv_h.shape[1]
    c = qkv_h.shape[2]
    kern = functools.partial(_causal_kernel, tq=tq, c=c)
    return pl.pallas_call(
        kern,
        grid=(h, t // tq),
        in_specs=[
            pl.BlockSpec((1, tq, c), lambda hh, i: (hh, i, 0)),
            pl.BlockSpec((1, t, c), lambda hh, i: (h + hh, 0, 0)),
            pl.BlockSpec((1, t, c), lambda hh, i: (2 * h + hh, 0, 0)),
        ],
        out_specs=pl.BlockSpec((1, tq, c), lambda hh, i: (hh, i, 0)),
        out_shape=jax.ShapeDtypeStruct((h, t, c), jnp.float32),
        compiler_params=pltpu.CompilerParams(
            dimension_semantics=("parallel", "parallel")),
    )(qkv_h, qkv_h, qkv_h)


def _knn_kernel(q_ref, memk_ref, topi_ref,
                v0, v1, v2, i0, i1, i2, *, mt, t):
    step = pl.program_id(0)
    nsteps = pl.num_programs(0)

    @pl.when(step == 0)
    def _init():
        neg = jnp.full((1, t), -jnp.inf, jnp.float32)
        zero = jnp.zeros((1, t), jnp.int32)
        v0[...] = neg
        v1[...] = neg
        v2[...] = neg
        i0[...] = zero
        i1[...] = zero
        i2[...] = zero

    k_tile = memk_ref[:, 0, :]
    s = lax.dot_general(k_tile, q_ref[...], (((1,), (1,)), ((), ())),
                        preferred_element_type=jnp.float32)
    rows = lax.broadcasted_iota(jnp.int32, (mt, t), 0)
    base = step * mt
    am = None
    for _ in range(3):
        if am is not None:
            s = jnp.where(rows == am, -jnp.inf, s)
        m = jnp.max(s, axis=0, keepdims=True)
        cand = jnp.where(s == m, rows, mt)
        am = jnp.min(cand, axis=0, keepdims=True)
        ci = am + base
        c0, c1, c2 = v0[...], v1[...], v2[...]
        j0, j1, j2 = i0[...], i1[...], i2[...]
        gt0 = m > c0
        gt1 = m > c1
        gt2 = m > c2
        v0[...] = jnp.where(gt0, m, c0)
        i0[...] = jnp.where(gt0, ci, j0)
        v1[...] = jnp.where(gt0, c0, jnp.where(gt1, m, c1))
        i1[...] = jnp.where(gt0, j0, jnp.where(gt1, ci, j1))
        v2[...] = jnp.where(gt1, c1, jnp.where(gt2, m, c2))
        i2[...] = jnp.where(gt1, j1, jnp.where(gt2, ci, j2))

    @pl.when(step == nsteps - 1)
    def _emit():
        topi_ref[...] = jnp.zeros((8, t), jnp.int32)
        topi_ref[0:1, :] = i0[...]
        topi_ref[1:2, :] = i1[...]
        topi_ref[2:3, :] = i2[...]


def _knn_top3(q_flat, mem3, mt=2048):
    t, ne = q_flat.shape
    m = mem3.shape[0]
    kern = functools.partial(_knn_kernel, mt=mt, t=t)
    return pl.pallas_call(
        kern,
        grid=(m // mt,),
        in_specs=[
            pl.BlockSpec((t, ne), lambda i: (0, 0)),
            pl.BlockSpec((mt, 2, ne), lambda i: (i, 0, 0)),
        ],
        out_specs=pl.BlockSpec((8, t), lambda i: (0, 0)),
        out_shape=jax.ShapeDtypeStruct((8, t), jnp.int32),
        scratch_shapes=[pltpu.VMEM((1, t), jnp.float32)] * 3
        + [pltpu.VMEM((1, t), jnp.int32)] * 3,
        compiler_params=pltpu.CompilerParams(
            dimension_semantics=("arbitrary",)),
    )(q_flat, mem3)


def _gather_rows(mem3, idx):
    mtot, two, d = mem3.shape
    btot = idx.shape[0]
    info = plsc.get_sparse_core_info()
    nw = info.num_cores * info.num_subcores
    b_per_w = btot // nw
    nchunks = 8
    ch = b_per_w // nchunks
    mesh = plsc.VectorSubcoreMesh(core_axis_name="c", subcore_axis_name="s")

    @functools.partial(
        pl.kernel,
        mesh=mesh,
        out_type=jax.ShapeDtypeStruct((btot, two, d), jnp.float32),
        scratch_types=[
            pltpu.VMEM((b_per_w,), jnp.int32),
            pltpu.VMEM((ch, two, d), jnp.float32),
            pltpu.VMEM((ch, two, d), jnp.float32),
            pltpu.SemaphoreType.DMA,
            pltpu.SemaphoreType.DMA,
        ],
    )
    def gather(mem_hbm, idx_hbm, out_hbm, idx_v, buf0, buf1, sem0, sem1):
        wid = lax.axis_index("s") * info.num_cores + lax.axis_index("c")
        base = wid * b_per_w
        pltpu.sync_copy(idx_hbm.at[pl.ds(base, b_per_w)], idx_v)
        bufs = (buf0, buf1)
        sems = (sem0, sem1)
        pending = [None] * nchunks
        for cidx in range(nchunks):
            pending[cidx] = pltpu.async_copy(
                mem_hbm.at[idx_v.at[pl.ds(cidx * ch, ch)]],
                bufs[cidx % 2], sems[cidx % 2])
            if cidx > 0:
                pending[cidx - 1].wait()
                pltpu.sync_copy(
                    bufs[(cidx - 1) % 2],
                    out_hbm.at[pl.ds(base + (cidx - 1) * ch, ch)])
        pending[nchunks - 1].wait()
        pltpu.sync_copy(
            bufs[(nchunks - 1) % 2],
            out_hbm.at[pl.ds(base + (nchunks - 1) * ch, ch)])

    return gather(mem3, idx)


def _combine_kernel(q_ref, y_ref, sel_ref, g_ref, wp_ref, o_ref,
                    *, scale, h, c):
    parts = []
    for hh in range(h):
        q = q_ref[hh]
        yh = y_ref[hh]
        logits = []
        for k in range(3):
            mk = sel_ref[k, :, 0, pl.ds(hh * c, c)]
            logits.append(jnp.sum(q * mk, axis=1, keepdims=True) * scale)
        l0, l1, l2 = logits
        mx = jnp.maximum(jnp.maximum(l0, l1), l2)
        e0 = jnp.exp(l0 - mx)
        e1 = jnp.exp(l1 - mx)
        e2 = jnp.exp(l2 - mx)
        denom = e0 + e1 + e2
        mem_qkv = (e0 * sel_ref[0, :, 1, pl.ds(hh * c, c)]
                   + e1 * sel_ref[1, :, 1, pl.ds(hh * c, c)]
                   + e2 * sel_ref[2, :, 1, pl.ds(hh * c, c)]) / denom
        g = g_ref[hh, 0]
        parts.append(mem_qkv * g + yh * (1.0 - g))
    combined = jnp.concatenate(parts, axis=1)
    o_ref[...] = jnp.dot(combined, wp_ref[...],
                         preferred_element_type=jnp.float32)


def _combine_proj(qkv_h, y, sel, gate, w_proj, h, c, scale, tq=256):
    t = y.shape[1]
    ne = h * c
    kern = functools.partial(_combine_kernel, scale=scale, h=h, c=c)
    return pl.pallas_call(
        kern,
        grid=(t // tq,),
        in_specs=[
            pl.BlockSpec((h, tq, c), lambda i: (0, i, 0)),
            pl.BlockSpec((h, tq, c), lambda i: (0, i, 0)),
            pl.BlockSpec((3, tq, 2, ne), lambda i: (0, i, 0, 0)),
            pl.BlockSpec(memory_space=pltpu.SMEM),
            pl.BlockSpec((ne, ne), lambda i: (0, 0)),
        ],
        out_specs=pl.BlockSpec((tq, ne), lambda i: (i, 0)),
        out_shape=jax.ShapeDtypeStruct((t, ne), jnp.float32),
        compiler_params=pltpu.CompilerParams(
            dimension_semantics=("parallel",)),
    )(qkv_h, y, sel, gate, w_proj)


def kernel(x, mem_kv, W_attn, W_proj, gate_bias):
    b, t, h, c = x.shape
    ne = h * c
    m = mem_kv.shape[1]
    xf = x.reshape(t, ne)
    w_heads = W_attn.reshape(ne, 3 * h, c).transpose(1, 0, 2)

    qkv_h = _qkv_proj(xf, w_heads)
    y = _causal_attn(qkv_h, h)
    q_flat = _q_flat(xf, W_attn)
    mem3 = mem_kv.reshape(m, 2, ne)
    topi = _knn_top3(q_flat, mem3)
    idx = topi[:3].reshape(3 * t)
    sel = _gather_rows(mem3, idx)
    sel = sel.reshape(3, t, 2, ne)
    scale = ne / (h ** (-0.5))
    gate = gate_bias.reshape(h, 1)
    out = _combine_proj(qkv_h, y, sel, gate, W_proj, h, c, scale)
    return out.reshape(b, t, ne)
